# Initial kernel scaffold; baseline (speedup 1.0000x reference)
import functools
import math

import jax
import jax.numpy as jnp
from jax import lax
from jax.experimental import pallas as pl
from jax.experimental.pallas import tpu as pltpu
from jax.experimental.pallas import tpu_sc as plsc

N = 10000
F = 128
E = 320000
POLY_D = 4
NUM_FILTERS = POLY_D + 1
NUM_TERMS = POLY_D + 3
K_STEPS = NUM_TERMS - 1

NUM_CORES = 2
NUM_SUBCORES = 16
NUM_WORKERS = NUM_CORES * NUM_SUBCORES
TN = 640
NP = NUM_SUBCORES * TN
CHUNK = 128
CPT = 79
EP = NUM_WORKERS * CPT * CHUNK
DUMMY = NP - 1
DW = 16


def _theta_coeffs(d):
    ev = 1.4
    offset = 2
    thetas = []
    for i in range(offset, d + 1 + offset):
        m = d - i + offset
        B = math.factorial(i) * math.factorial(d + 2 - i) / math.factorial(d + 3)
        coeffs = [0.0] * (d + offset + 1)
        for j in range(m + 1):
            coeffs[i + j] = math.comb(m, j) * ((-1.0 / ev) ** j) / (ev ** i) / (ev * B)
        thetas.append(coeffs)
    return thetas


THETAS = _theta_coeffs(POLY_D)

_MESH = dict(core_axis_name="c", subcore_axis_name="s",
             num_cores=NUM_CORES, num_subcores=NUM_SUBCORES)


def _sc_degree(src_r):
    @functools.partial(
        pl.kernel,
        out_type=jax.ShapeDtypeStruct((NUM_CORES * NP, DW), jnp.float32),
        mesh=plsc.VectorSubcoreMesh(**_MESH),
        scratch_types=[
            pltpu.VMEM((CPT, CHUNK), jnp.int32),
            pltpu.VMEM((CHUNK, DW), jnp.float32),
            pltpu.VMEM((TN, DW), jnp.float32),
            pltpu.VMEM_SHARED((NP, DW), jnp.float32),
        ],
    )
    def deg_kernel(src_hbm, out_hbm, idx_v, ones_v, stage_v, acc_sh):
        cid = lax.axis_index("c")
        sid = lax.axis_index("s")
        wid = sid * NUM_CORES + cid

        one = jnp.full((DW,), 1.0, jnp.float32)
        zero = jnp.zeros((DW,), jnp.float32)

        def fill_ones(i, carry):
            ones_v[i, :] = one
            return carry

        lax.fori_loop(0, CHUNK, fill_ones, 0)

        def fill_zero(i, carry):
            stage_v[i, :] = zero
            return carry

        lax.fori_loop(0, TN, fill_zero, 0)

        pltpu.sync_copy(stage_v, acc_sh.at[pl.ds(sid * TN, TN)])
        plsc.subcore_barrier()

        pltpu.sync_copy(src_hbm.at[wid], idx_v)

        def body(j, carry):
            pltpu.sync_copy(ones_v, acc_sh.at[idx_v.at[j]], add=True)
            return carry

        lax.fori_loop(0, CPT, body, 0)

        plsc.subcore_barrier()
        pltpu.sync_copy(acc_sh.at[pl.ds(sid * TN, TN)], stage_v)
        pltpu.sync_copy(stage_v, out_hbm.at[pl.ds(cid * NP + sid * TN, TN)])

    return deg_kernel(src_r)


def _sc_spmm(g_pad, src_r, dst_r):
    @functools.partial(
        pl.kernel,
        out_type=jax.ShapeDtypeStruct((NUM_CORES * NP, F), jnp.float32),
        mesh=plsc.VectorSubcoreMesh(**_MESH),
        scratch_types=[
            pltpu.VMEM((CPT, CHUNK), jnp.int32),
            pltpu.VMEM((CPT, CHUNK), jnp.int32),
            pltpu.VMEM((CHUNK, F), jnp.float32),
            pltpu.VMEM_SHARED((NP, F), jnp.float32),
            pltpu.SemaphoreType.DMA,
        ],
    )
    def spmm_kernel(g_hbm, src_hbm, dst_hbm, out_hbm,
                    src_v, dst_v, rows_v, acc_sh, sem):
        cid = lax.axis_index("c")
        sid = lax.axis_index("s")
        wid = sid * NUM_CORES + cid

        zero = jnp.zeros((16,), jnp.float32)

        def fill_zero(i, carry):
            for l in range(F // 16):
                rows_v[i, pl.ds(l * 16, 16)] = zero
            return carry

        lax.fori_loop(0, CHUNK, fill_zero, 0)

        for t in range(TN // CHUNK):
            pltpu.sync_copy(rows_v, acc_sh.at[pl.ds(sid * TN + t * CHUNK, CHUNK)])
        plsc.subcore_barrier()

        pltpu.sync_copy(src_hbm.at[wid], src_v)
        pltpu.sync_copy(dst_hbm.at[wid], dst_v)

        def body(j, carry):
            pltpu.async_copy(g_hbm.at[src_v.at[j]], rows_v, sem).wait()
            pltpu.sync_copy(rows_v, acc_sh.at[dst_v.at[j]], add=True)
            return carry

        lax.fori_loop(0, CPT, body, 0)

        plsc.subcore_barrier()
        for t in range(TN // CHUNK):
            pltpu.sync_copy(acc_sh.at[pl.ds(sid * TN + t * CHUNK, CHUNK)], rows_v)
            pltpu.sync_copy(
                rows_v, out_hbm.at[pl.ds(cid * NP + sid * TN + t * CHUNK, CHUNK)])

    return spmm_kernel(g_pad, src_r, dst_r)


def _tc_head(x_p, W, b2, degp):
    def body(x_ref, w_ref, b_ref, d0_ref, d1_ref, p0_ref, g0_ref, ds_ref):
        h = jnp.dot(x_ref[...], w_ref[...], preferred_element_type=jnp.float32)
        h = h + b_ref[...]
        h = jnp.where(h >= 0.0, h, 0.01 * h)
        deg = d0_ref[...] + d1_ref[...]
        dsv = lax.rsqrt(jnp.maximum(deg, 1.0))
        ds_ref[...] = dsv
        p0_ref[...] = h
        g0_ref[...] = dsv[:, :1] * h

    return pl.pallas_call(
        body,
        grid=(NP // TN,),
        in_specs=[
            pl.BlockSpec((TN, F), lambda j: (j, 0)),
            pl.BlockSpec((F, F), lambda j: (0, 0)),
            pl.BlockSpec((1, F), lambda j: (0, 0)),
            pl.BlockSpec((TN, DW), lambda j: (j, 0)),
            pl.BlockSpec((TN, DW), lambda j: (j + NUM_SUBCORES, 0)),
        ],
        out_specs=[
            pl.BlockSpec((TN, F), lambda j: (j, 0)),
            pl.BlockSpec((TN, F), lambda j: (j, 0)),
            pl.BlockSpec((TN, DW), lambda j: (j, 0)),
        ],
        out_shape=[
            jax.ShapeDtypeStruct((NP, F), jnp.float32),
            jax.ShapeDtypeStruct((NP, F), jnp.float32),
            jax.ShapeDtypeStruct((NP, DW), jnp.float32),
        ],
    )(x_p, W, b2, degp, degp)


def _tc_update(p, aggp, ds):
    def body(p_ref, a0_ref, a1_ref, ds_ref, pn_ref, gn_ref):
        agg = a0_ref[...] + a1_ref[...]
        dsv = ds_ref[...][:, :1]
        pn = p_ref[...] - dsv * agg
        pn_ref[...] = pn
        gn_ref[...] = dsv * pn

    return pl.pallas_call(
        body,
        grid=(NP // TN,),
        in_specs=[
            pl.BlockSpec((TN, F), lambda j: (j, 0)),
            pl.BlockSpec((TN, F), lambda j: (j, 0)),
            pl.BlockSpec((TN, F), lambda j: (j + NUM_SUBCORES, 0)),
            pl.BlockSpec((TN, DW), lambda j: (j, 0)),
        ],
        out_specs=[
            pl.BlockSpec((TN, F), lambda j: (j, 0)),
            pl.BlockSpec((TN, F), lambda j: (j, 0)),
        ],
        out_shape=[
            jax.ShapeDtypeStruct((NP, F), jnp.float32),
            jax.ShapeDtypeStruct((NP, F), jnp.float32),
        ],
    )(p, aggp, aggp, ds)


def _tc_combine(ps):
    ROWS = 1000

    def body(*refs):
        p_refs = refs[:NUM_TERMS]
        out_ref = refs[NUM_TERMS]
        vals = [r[...] for r in p_refs]
        for i in range(NUM_FILTERS):
            acc = THETAS[i][0] * vals[0]
            for k in range(1, NUM_TERMS):
                acc = acc + THETAS[i][k] * vals[k]
            out_ref[:, i * F:(i + 1) * F] = acc

    return pl.pallas_call(
        body,
        grid=(N // ROWS,),
        in_specs=[pl.BlockSpec((ROWS, F), lambda j: (j, 0))] * NUM_TERMS,
        out_specs=pl.BlockSpec((ROWS, NUM_FILTERS * F), lambda j: (j, 0)),
        out_shape=jax.ShapeDtypeStruct((N, NUM_FILTERS * F), jnp.float32),
    )(*ps)


def kernel(x, edge_index, W, b):
    src = edge_index[0].astype(jnp.int32)
    dst = edge_index[1].astype(jnp.int32)
    pad_idx = jnp.full((EP - E,), DUMMY, jnp.int32)
    src_r = jnp.concatenate([src, pad_idx]).reshape(NUM_WORKERS, CPT, CHUNK)
    dst_r = jnp.concatenate([dst, pad_idx]).reshape(NUM_WORKERS, CPT, CHUNK)
    x_p = jnp.pad(x, ((0, NP - N), (0, 0)))
    b2 = b.reshape(1, F)

    degp = _sc_degree(src_r)
    p0, g0, ds = _tc_head(x_p, W, b2, degp)

    ps = [p0]
    g = g0
    for _ in range(K_STEPS):
        aggp = _sc_spmm(g, src_r, dst_r)
        pn, g = _tc_update(ps[-1], aggp, ds)
        ps.append(pn)
    return _tc_combine(ps)


# speedup vs baseline: 3.4337x; 3.4337x over previous
import functools
import math

import jax
import jax.numpy as jnp
from jax import lax
from jax.experimental import pallas as pl
from jax.experimental.pallas import tpu as pltpu
from jax.experimental.pallas import tpu_sc as plsc

N = 10000
F = 128
E = 320000
POLY_D = 4
NUM_FILTERS = POLY_D + 1
NUM_TERMS = POLY_D + 3
K_STEPS = NUM_TERMS - 1

NUM_CORES = 2
NUM_SUBCORES = 16
NUM_WORKERS = NUM_CORES * NUM_SUBCORES
TN = 640
NP = NUM_SUBCORES * TN
CHUNK = 128
CPT = 79
EPW = CPT * CHUNK
EP = NUM_WORKERS * EPW
DUMMY = NP - 1
DW = 16


def _theta_coeffs(d):
    ev = 1.4
    offset = 2
    thetas = []
    for i in range(offset, d + 1 + offset):
        m = d - i + offset
        B = math.factorial(i) * math.factorial(d + 2 - i) / math.factorial(d + 3)
        coeffs = [0.0] * (d + offset + 1)
        for j in range(m + 1):
            coeffs[i + j] = math.comb(m, j) * ((-1.0 / ev) ** j) / (ev ** i) / (ev * B)
        thetas.append(coeffs)
    return thetas


THETAS = _theta_coeffs(POLY_D)

_MESH = dict(core_axis_name="c", subcore_axis_name="s",
             num_cores=NUM_CORES, num_subcores=NUM_SUBCORES)


def _sc_degree(src_flat):
    @functools.partial(
        pl.kernel,
        out_type=jax.ShapeDtypeStruct((NUM_CORES * NP, F), jnp.float32),
        mesh=plsc.VectorSubcoreMesh(**_MESH),
        scratch_types=[
            pltpu.VMEM((CHUNK,), jnp.int32),
            pltpu.VMEM((CHUNK, F), jnp.float32),
            pltpu.VMEM_SHARED((NP, F), jnp.float32),
        ],
    )
    def deg_kernel(src_hbm, out_hbm, idx_v, ones_v, acc_sh):
        cid = lax.axis_index("c")
        sid = lax.axis_index("s")
        wid = sid * NUM_CORES + cid
        ebase = wid * EPW

        one = jnp.full((16,), 1.0, jnp.float32)
        zero = jnp.zeros((16,), jnp.float32)

        def zfill(i, carry):
            for l in range(F // 16):
                ones_v[i, pl.ds(l * 16, 16)] = zero
            return carry

        def fill(i, carry):
            for l in range(F // 16):
                ones_v[i, pl.ds(l * 16, 16)] = one
            return carry

        lax.fori_loop(0, CHUNK, zfill, 0)
        for t in range(TN // CHUNK):
            pltpu.sync_copy(ones_v, acc_sh.at[pl.ds(sid * TN + t * CHUNK, CHUNK)])
        lax.fori_loop(0, CHUNK, fill, 0)
        plsc.subcore_barrier()

        def body(j, carry):
            pltpu.sync_copy(src_hbm.at[pl.ds(ebase + j * CHUNK, CHUNK)], idx_v)
            pltpu.sync_copy(ones_v, acc_sh.at[idx_v], add=True)
            return carry

        lax.fori_loop(0, CPT, body, 0)
        plsc.subcore_barrier()
        for t in range(TN // CHUNK):
            pltpu.sync_copy(acc_sh.at[pl.ds(sid * TN + t * CHUNK, CHUNK)], ones_v)
            pltpu.sync_copy(
                ones_v, out_hbm.at[pl.ds(cid * NP + sid * TN + t * CHUNK, CHUNK)])

    return deg_kernel(src_flat)


def _sc_spmm(g_pad, src_flat, dst_flat):
    @functools.partial(
        pl.kernel,
        out_type=jax.ShapeDtypeStruct((NUM_CORES * NP, F), jnp.float32),
        mesh=plsc.VectorSubcoreMesh(**_MESH),
        scratch_types=[
            pltpu.VMEM((CHUNK,), jnp.int32),
            pltpu.VMEM((CHUNK,), jnp.int32),
            pltpu.VMEM((CHUNK, F), jnp.float32),
            pltpu.VMEM_SHARED((NP, F), jnp.float32),
            pltpu.SemaphoreType.DMA,
        ],
    )
    def spmm_kernel(g_hbm, src_hbm, dst_hbm, out_hbm,
                    srci_v, dsti_v, rows_v, acc_sh, sem):
        cid = lax.axis_index("c")
        sid = lax.axis_index("s")
        wid = sid * NUM_CORES + cid
        ebase = wid * EPW

        zero = jnp.zeros((16,), jnp.float32)

        def zfill(i, carry):
            for l in range(F // 16):
                rows_v[i, pl.ds(l * 16, 16)] = zero
            return carry

        lax.fori_loop(0, CHUNK, zfill, 0)
        for t in range(TN // CHUNK):
            pltpu.sync_copy(rows_v, acc_sh.at[pl.ds(sid * TN + t * CHUNK, CHUNK)])
        plsc.subcore_barrier()

        def body(j, carry):
            pltpu.sync_copy(src_hbm.at[pl.ds(ebase + j * CHUNK, CHUNK)], srci_v)
            pltpu.sync_copy(dst_hbm.at[pl.ds(ebase + j * CHUNK, CHUNK)], dsti_v)
            pltpu.async_copy(g_hbm.at[srci_v], rows_v, sem).wait()
            pltpu.sync_copy(rows_v, acc_sh.at[dsti_v], add=True)
            return carry

        lax.fori_loop(0, CPT, body, 0)

        plsc.subcore_barrier()
        for t in range(TN // CHUNK):
            pltpu.sync_copy(acc_sh.at[pl.ds(sid * TN + t * CHUNK, CHUNK)], rows_v)
            pltpu.sync_copy(
                rows_v, out_hbm.at[pl.ds(cid * NP + sid * TN + t * CHUNK, CHUNK)])

    return spmm_kernel(g_pad, src_flat, dst_flat)


def _tc_head(x_p, W, b2, degp):
    def body(x_ref, w_ref, b_ref, d0_ref, d1_ref, p0_ref, g0_ref, ds_ref):
        h = jnp.dot(x_ref[...], w_ref[...], preferred_element_type=jnp.float32)
        h = h + b_ref[...]
        h = jnp.where(h >= 0.0, h, 0.01 * h)
        deg = d0_ref[...][:, :DW] + d1_ref[...][:, :DW]
        dsv = lax.rsqrt(jnp.maximum(deg, 1.0))
        ds_ref[...] = dsv
        p0_ref[...] = h
        g0_ref[...] = dsv[:, :1] * h

    return pl.pallas_call(
        body,
        grid=(NP // TN,),
        in_specs=[
            pl.BlockSpec((TN, F), lambda j: (j, 0)),
            pl.BlockSpec((F, F), lambda j: (0, 0)),
            pl.BlockSpec((1, F), lambda j: (0, 0)),
            pl.BlockSpec((TN, F), lambda j: (j, 0)),
            pl.BlockSpec((TN, F), lambda j: (j + NUM_SUBCORES, 0)),
        ],
        out_specs=[
            pl.BlockSpec((TN, F), lambda j: (j, 0)),
            pl.BlockSpec((TN, F), lambda j: (j, 0)),
            pl.BlockSpec((TN, DW), lambda j: (j, 0)),
        ],
        out_shape=[
            jax.ShapeDtypeStruct((NP, F), jnp.float32),
            jax.ShapeDtypeStruct((NP, F), jnp.float32),
            jax.ShapeDtypeStruct((NP, DW), jnp.float32),
        ],
    )(x_p, W, b2, degp, degp)


def _tc_update(p, aggp, ds):
    def body(p_ref, a0_ref, a1_ref, ds_ref, pn_ref, gn_ref):
        agg = a0_ref[...] + a1_ref[...]
        dsv = ds_ref[...][:, :1]
        pn = p_ref[...] - dsv * agg
        pn_ref[...] = pn
        gn_ref[...] = dsv * pn

    return pl.pallas_call(
        body,
        grid=(NP // TN,),
        in_specs=[
            pl.BlockSpec((TN, F), lambda j: (j, 0)),
            pl.BlockSpec((TN, F), lambda j: (j, 0)),
            pl.BlockSpec((TN, F), lambda j: (j + NUM_SUBCORES, 0)),
            pl.BlockSpec((TN, DW), lambda j: (j, 0)),
        ],
        out_specs=[
            pl.BlockSpec((TN, F), lambda j: (j, 0)),
            pl.BlockSpec((TN, F), lambda j: (j, 0)),
        ],
        out_shape=[
            jax.ShapeDtypeStruct((NP, F), jnp.float32),
            jax.ShapeDtypeStruct((NP, F), jnp.float32),
        ],
    )(p, aggp, aggp, ds)


def _tc_combine(ps):
    ROWS = 1000

    def body(*refs):
        p_refs = refs[:NUM_TERMS]
        out_ref = refs[NUM_TERMS]
        vals = [r[...] for r in p_refs]
        for i in range(NUM_FILTERS):
            acc = THETAS[i][0] * vals[0]
            for k in range(1, NUM_TERMS):
                acc = acc + THETAS[i][k] * vals[k]
            out_ref[:, i * F:(i + 1) * F] = acc

    return pl.pallas_call(
        body,
        grid=(N // ROWS,),
        in_specs=[pl.BlockSpec((ROWS, F), lambda j: (j, 0))] * NUM_TERMS,
        out_specs=pl.BlockSpec((ROWS, NUM_FILTERS * F), lambda j: (j, 0)),
        out_shape=jax.ShapeDtypeStruct((N, NUM_FILTERS * F), jnp.float32),
    )(*ps)


def kernel(x, edge_index, W, b):
    src = edge_index[0].astype(jnp.int32)
    dst = edge_index[1].astype(jnp.int32)
    pad_idx = jnp.full((EP - E,), DUMMY, jnp.int32)
    src_flat = jnp.concatenate([src, pad_idx])
    dst_flat = jnp.concatenate([dst, pad_idx])
    x_p = jnp.pad(x, ((0, NP - N), (0, 0)))
    b2 = b.reshape(1, F)

    degp = _sc_degree(src_flat)
    p0, g0, ds = _tc_head(x_p, W, b2, degp)

    ps = [p0]
    g = g0
    for _ in range(K_STEPS):
        aggp = _sc_spmm(g, src_flat, dst_flat)
        pn, g = _tc_update(ps[-1], aggp, ds)
        ps.append(pn)
    return _tc_combine(ps)
